# Initial kernel scaffold; baseline (speedup 1.0000x reference)
import jax
import jax.numpy as jnp
from jax.experimental import pallas as pl


def kernel(indexes, table, W):
    raise NotImplementedError("write your pallas kernel here")


# speedup vs baseline: 1.1625x; 1.1625x over previous
import functools

import jax
import jax.numpy as jnp
from jax import lax
from jax.experimental import pallas as pl
from jax.experimental.pallas import tpu as pltpu
from jax.experimental.pallas import tpu_sc as plsc

B = 16384
L = 50
FLAT = B * L
D = 64
P = 64

NC = 2
NS = 16
NW = NC * NS
G = 128
GROUPS = FLAT // G
GPW = GROUPS // NW
INNER = 8
OUTER = GPW // INNER


def _sc_gather(idx2d, table):
    mesh = plsc.VectorSubcoreMesh(core_axis_name="c", subcore_axis_name="s")

    @functools.partial(
        pl.kernel,
        out_type=jax.ShapeDtypeStruct((FLAT, D), jnp.float32),
        mesh=mesh,
        compiler_params=pltpu.CompilerParams(use_tc_tiling_on_sc=False),
        scratch_types=[
            pltpu.VMEM((INNER, G), jnp.int32),
            pltpu.VMEM((INNER * G, D), jnp.float32),
            pltpu.SemaphoreType.DMA,
        ],
    )
    def k(idx_hbm, table_hbm, out_hbm, idx_v, rows_v, sem):
        wid = lax.axis_index("s") * NC + lax.axis_index("c")
        g_base = wid * GPW

        def body(c, _):
            g0 = g_base + c * INNER
            pltpu.sync_copy(idx_hbm.at[pl.ds(g0, INNER)], idx_v)
            copies = [
                pltpu.async_copy(
                    table_hbm.at[idx_v.at[j]],
                    rows_v.at[pl.ds(j * G, G)],
                    sem,
                )
                for j in range(INNER)
            ]
            for cp in copies:
                cp.wait()
            pltpu.sync_copy(rows_v, out_hbm.at[pl.ds(g0 * G, INNER * G)])
            return _

        lax.fori_loop(0, OUTER, body, None)

    return k(idx2d, table)


def _tc_project(emb, W):
    BLK = 4096

    def body(e_ref, w_ref, o_ref):
        o_ref[...] = lax.dot_general(
            e_ref[...], w_ref[...],
            dimension_numbers=(((1,), (1,)), ((), ())),
            preferred_element_type=jnp.float32,
        )

    return pl.pallas_call(
        body,
        grid=(FLAT // BLK,),
        in_specs=[
            pl.BlockSpec((BLK, D), lambda i: (i, 0)),
            pl.BlockSpec((P, D), lambda i: (0, 0)),
        ],
        out_specs=pl.BlockSpec((BLK, P), lambda i: (i, 0)),
        out_shape=jax.ShapeDtypeStruct((FLAT, P), jnp.float32),
    )(emb, W)


def kernel(indexes, table, W):
    idx2d = indexes.astype(jnp.int32).reshape(GROUPS, G)
    emb = _sc_gather(idx2d, table)
    out = _tc_project(emb, W)
    return out.reshape(B, L, P)


# speedup vs baseline: 2.1448x; 1.8449x over previous
import functools

import jax
import jax.numpy as jnp
from jax import lax
from jax.experimental import pallas as pl
from jax.experimental.pallas import tpu as pltpu
from jax.experimental.pallas import tpu_sc as plsc

B = 16384
L = 50
FLAT = B * L
D = 64
P = 64

NC = 2
NS = 16
NW = NC * NS
G = 128
GROUPS = FLAT // G
GPW = GROUPS // NW
INNER = 8
OUTER = GPW // INNER


def _sc_gather(idx2d, table):
    mesh = plsc.VectorSubcoreMesh(core_axis_name="c", subcore_axis_name="s")

    @functools.partial(
        pl.kernel,
        out_type=jax.ShapeDtypeStruct((FLAT, D), jnp.float32),
        mesh=mesh,
        compiler_params=pltpu.CompilerParams(use_tc_tiling_on_sc=False),
        scratch_types=[
            pltpu.VMEM((INNER, G), jnp.int32),
            pltpu.VMEM((INNER * G, D), jnp.float32),
            pltpu.SemaphoreType.DMA,
        ],
    )
    def k(idx_hbm, table_hbm, out_hbm, idx_v, rows_v, sem):
        wid = lax.axis_index("s") * NC + lax.axis_index("c")
        g_base = wid * GPW

        def body(c, _):
            g0 = g_base + c * INNER
            pltpu.sync_copy(idx_hbm.at[pl.ds(g0, INNER)], idx_v)
            copies = [
                pltpu.async_copy(
                    table_hbm.at[idx_v.at[j]],
                    rows_v.at[pl.ds(j * G, G)],
                    sem,
                )
                for j in range(INNER)
            ]
            for cp in copies:
                cp.wait()
            pltpu.sync_copy(rows_v, out_hbm.at[pl.ds(g0 * G, INNER * G)])
            return _

        lax.fori_loop(0, OUTER, body, None)

    return k(idx2d, table)


def _tc_project_t(e128, W2):
    H = B // 2

    def body(e_ref, w_ref, o_ref):
        r2 = lax.dot_general(
            w_ref[...], e_ref[...],
            dimension_numbers=(((1,), (1,)), ((), ())),
            preferred_element_type=jnp.float32,
        )
        o_ref[0, :, :H] = r2[:P, :]
        o_ref[0, :, H:] = r2[P:, :]

    return pl.pallas_call(
        body,
        grid=(L,),
        in_specs=[
            pl.BlockSpec((H, 2 * D), lambda i: (i, 0)),
            pl.BlockSpec((2 * P, 2 * D), lambda i: (0, 0)),
        ],
        out_specs=pl.BlockSpec((1, P, B), lambda i: (i, 0, 0)),
        out_shape=jax.ShapeDtypeStruct((L, P, B), jnp.float32),
    )(e128, W2)


def kernel(indexes, table, W):
    idxg = (indexes.astype(jnp.int32).T
            .reshape(L, 2, B // 2).transpose(0, 2, 1).reshape(GROUPS, G))
    emb = _sc_gather(idxg, table)
    e128 = emb.reshape(FLAT // 2, 2 * D)
    W2 = jnp.zeros((2 * P, 2 * D), jnp.float32)
    W2 = W2.at[:P, :D].set(W).at[P:, D:].set(W)
    outT = _tc_project_t(e128, W2)
    return outT.transpose(2, 0, 1)


# speedup vs baseline: 2.8057x; 1.3081x over previous
import functools

import jax
import jax.numpy as jnp
from jax import lax
from jax.experimental import pallas as pl
from jax.experimental.pallas import tpu as pltpu
from jax.experimental.pallas import tpu_sc as plsc

B = 16384
L = 50
FLAT = B * L
D = 64
P = 64

NC = 2
NS = 16
NW = NC * NS
G = 128
GROUPS = FLAT // G
GPW = GROUPS // NW
INNER = 8
OUTER = GPW // INNER


def _sc_gather(idx2d, table):
    mesh = plsc.VectorSubcoreMesh(core_axis_name="c", subcore_axis_name="s")

    @functools.partial(
        pl.kernel,
        out_type=jax.ShapeDtypeStruct((FLAT, D), jnp.float32),
        mesh=mesh,
        compiler_params=pltpu.CompilerParams(use_tc_tiling_on_sc=False),
        scratch_types=[
            pltpu.VMEM((INNER, G), jnp.int32),
            pltpu.VMEM((INNER * G, D), jnp.float32),
            pltpu.SemaphoreType.DMA,
        ],
    )
    def k(idx_hbm, table_hbm, out_hbm, idx_v, rows_v, sem):
        wid = lax.axis_index("s") * NC + lax.axis_index("c")
        g_base = wid * GPW

        def body(c, _):
            g0 = g_base + c * INNER
            pltpu.sync_copy(idx_hbm.at[pl.ds(g0, INNER)], idx_v)
            copies = [
                pltpu.async_copy(
                    table_hbm.at[idx_v.at[j]],
                    rows_v.at[pl.ds(j * G, G)],
                    sem,
                )
                for j in range(INNER)
            ]
            for cp in copies:
                cp.wait()
            pltpu.sync_copy(rows_v, out_hbm.at[pl.ds(g0 * G, INNER * G)])
            return _

        lax.fori_loop(0, OUTER, body, None)

    return k(idx2d, table)


SPLIT = 524288
NUMR = 1000000
PACK_BLK = 4096
PACK_NB = SPLIT // PACK_BLK
MAX_LANE_BLK = (NUMR + PACK_BLK - 1) // PACK_BLK - 1


def _tc_pack(tableT, ident):
    def body(lo_ref, hi_ref, i_ref, o_ref):
        tlo = lax.dot_general(
            lo_ref[...], i_ref[...],
            dimension_numbers=(((0,), (0,)), ((), ())),
            preferred_element_type=jnp.float32,
        )
        thi = lax.dot_general(
            hi_ref[...], i_ref[...],
            dimension_numbers=(((0,), (0,)), ((), ())),
            preferred_element_type=jnp.float32,
        )
        o_ref[:, :D] = tlo
        o_ref[:, D:] = thi

    return pl.pallas_call(
        body,
        grid=(PACK_NB,),
        in_specs=[
            pl.BlockSpec((D, PACK_BLK), lambda i: (0, i)),
            pl.BlockSpec((D, PACK_BLK),
                         lambda i: (0, jnp.minimum(PACK_NB + i, MAX_LANE_BLK))),
            pl.BlockSpec((D, D), lambda i: (0, 0)),
        ],
        out_specs=pl.BlockSpec((PACK_BLK, 2 * D), lambda i: (i, 0)),
        out_shape=jax.ShapeDtypeStruct((SPLIT, 2 * D), jnp.float32),
    )(tableT, tableT, ident)


def _tc_project_t(e128, W2):
    H = B // 2

    def body(e_ref, w_ref, o_ref):
        r2 = lax.dot_general(
            w_ref[...], e_ref[...],
            dimension_numbers=(((1,), (1,)), ((), ())),
            preferred_element_type=jnp.float32,
        )
        o_ref[0, :, :H] = r2[:P, :]
        o_ref[0, :, H:] = r2[P:, :]

    return pl.pallas_call(
        body,
        grid=(L,),
        in_specs=[
            pl.BlockSpec((H, 2 * D), lambda i: (i, 0)),
            pl.BlockSpec((2 * P, 2 * D), lambda i: (0, 0)),
        ],
        out_specs=pl.BlockSpec((1, P, B), lambda i: (i, 0, 0)),
        out_shape=jax.ShapeDtypeStruct((L, P, B), jnp.float32),
    )(e128, W2)


def kernel(indexes, table, W):
    idxg = (indexes.astype(jnp.int32).T
            .reshape(L, 2, B // 2).transpose(0, 2, 1).reshape(GROUPS, G))
    idxg = jnp.where(idxg < SPLIT, idxg * 2, (idxg - SPLIT) * 2 + 1)
    packed = _tc_pack(table.T, jnp.eye(D, dtype=jnp.float32))
    pview = packed.reshape(2 * SPLIT, D)
    emb = _sc_gather(idxg, pview)
    e128 = emb.reshape(FLAT // 2, 2 * D)
    W2 = jnp.zeros((2 * P, 2 * D), jnp.float32)
    W2 = W2.at[:P, :D].set(W).at[P:, D:].set(W)
    outT = _tc_project_t(e128, W2)
    return outT.transpose(2, 0, 1)


# speedup vs baseline: 3.8740x; 1.3808x over previous
import functools

import jax
import jax.numpy as jnp
from jax import lax
from jax.experimental import pallas as pl
from jax.experimental.pallas import tpu as pltpu
from jax.experimental.pallas import tpu_sc as plsc

B = 16384
L = 50
FLAT = B * L
D = 64
P = 64

NC = 2
NS = 16
NW = NC * NS
G = 128
GROUPS = FLAT // G
GPW = GROUPS // NW
INNER = 8
OUTER = GPW // INNER


ROWS_PER_IT = INNER * G
HALF_IT = ROWS_PER_IT // 2


def _sc_gather(idxt, table):
    mesh = plsc.VectorSubcoreMesh(core_axis_name="c", subcore_axis_name="s")

    @functools.partial(
        pl.kernel,
        out_type=jax.ShapeDtypeStruct((FLAT, D), jnp.float32),
        mesh=mesh,
        compiler_params=pltpu.CompilerParams(use_tc_tiling_on_sc=False),
        scratch_types=[
            pltpu.VMEM((ROWS_PER_IT,), jnp.int32),
            pltpu.VMEM((ROWS_PER_IT,), jnp.int32),
            pltpu.VMEM((ROWS_PER_IT, D), jnp.float32),
            pltpu.SemaphoreType.DMA,
        ],
    )
    def k(idx_hbm, table_hbm, out_hbm, idx_ab, idx_v, rows_v, sem):
        wid = lax.axis_index("s") * NC + lax.axis_index("c")
        r_base = wid * (FLAT // NW)

        def body(c, _):
            lane = lax.iota(jnp.int32, 16)
            q0 = lax.shift_right_logical(lane, 1)
            evenm = (lane & 1) == 0
            r0 = r_base + c * ROWS_PER_IT
            l = lax.shift_right_logical(r0, 14)
            k0 = pl.multiple_of(
                lax.shift_right_logical(r0 - l * B, 1), HALF_IT)
            pltpu.sync_copy(idx_hbm.at[l, pl.ds(k0, HALF_IT)],
                            idx_ab.at[pl.ds(0, HALF_IT)])
            pltpu.sync_copy(idx_hbm.at[l, pl.ds(B // 2 + k0, HALF_IT)],
                            idx_ab.at[pl.ds(HALF_IT, HALF_IT)])
            for m in range(HALF_IT // 16):
                a = idx_ab[pl.ds(m * 16, 16)]
                b = idx_ab[pl.ds(HALF_IT + m * 16, 16)]
                lo = jnp.where(evenm, a.at[q0].get(mode='promise_in_bounds'),
                               b.at[q0].get(mode='promise_in_bounds'))
                hi = jnp.where(evenm,
                               a.at[q0 + 8].get(mode='promise_in_bounds'),
                               b.at[q0 + 8].get(mode='promise_in_bounds'))
                idx_v[pl.ds(m * 32, 16)] = lo
                idx_v[pl.ds(m * 32 + 16, 16)] = hi
            copies = [
                pltpu.async_copy(
                    table_hbm.at[idx_v.at[pl.ds(j * G, G)]],
                    rows_v.at[pl.ds(j * G, G)],
                    sem,
                )
                for j in range(INNER)
            ]
            for cp in copies:
                cp.wait()
            pltpu.sync_copy(rows_v, out_hbm.at[pl.ds(r0, ROWS_PER_IT)])
            return _

        lax.fori_loop(0, OUTER, body, None)

    return k(idxt, table)


SPLIT = 524288
NUMR = 1000000
PACK_BLK = 4096
PACK_NB = SPLIT // PACK_BLK
MAX_LANE_BLK = (NUMR + PACK_BLK - 1) // PACK_BLK - 1


def _tc_pack(tableT, ident):
    def body(lo_ref, hi_ref, i_ref, o_ref):
        tlo = lax.dot_general(
            lo_ref[...], i_ref[...],
            dimension_numbers=(((0,), (0,)), ((), ())),
            preferred_element_type=jnp.float32,
        )
        thi = lax.dot_general(
            hi_ref[...], i_ref[...],
            dimension_numbers=(((0,), (0,)), ((), ())),
            preferred_element_type=jnp.float32,
        )
        o_ref[:, :D] = tlo
        o_ref[:, D:] = thi

    return pl.pallas_call(
        body,
        grid=(PACK_NB,),
        in_specs=[
            pl.BlockSpec((D, PACK_BLK), lambda i: (0, i)),
            pl.BlockSpec((D, PACK_BLK),
                         lambda i: (0, jnp.minimum(PACK_NB + i, MAX_LANE_BLK))),
            pl.BlockSpec((D, D), lambda i: (0, 0)),
        ],
        out_specs=pl.BlockSpec((PACK_BLK, 2 * D), lambda i: (i, 0)),
        out_shape=jax.ShapeDtypeStruct((SPLIT, 2 * D), jnp.float32),
    )(tableT, tableT, ident)


def _tc_project_t(e128, W2):
    H = B // 2

    def body(e_ref, w_ref, o_ref):
        r2 = lax.dot_general(
            w_ref[...], e_ref[...],
            dimension_numbers=(((1,), (1,)), ((), ())),
            preferred_element_type=jnp.float32,
        )
        o_ref[0, :, :H] = r2[:P, :]
        o_ref[0, :, H:] = r2[P:, :]

    return pl.pallas_call(
        body,
        grid=(L,),
        in_specs=[
            pl.BlockSpec((H, 2 * D), lambda i: (i, 0)),
            pl.BlockSpec((2 * P, 2 * D), lambda i: (0, 0)),
        ],
        out_specs=pl.BlockSpec((1, P, B), lambda i: (i, 0, 0)),
        out_shape=jax.ShapeDtypeStruct((L, P, B), jnp.float32),
    )(e128, W2)


def kernel(indexes, table, W):
    idxt = indexes.astype(jnp.int32).T
    idxt = jnp.where(idxt < SPLIT, idxt * 2, (idxt - SPLIT) * 2 + 1)
    packed = _tc_pack(table.T, jnp.eye(D, dtype=jnp.float32))
    pview = packed.reshape(2 * SPLIT, D)
    emb = _sc_gather(idxt, pview)
    e128 = emb.reshape(FLAT // 2, 2 * D)
    W2 = jnp.zeros((2 * P, 2 * D), jnp.float32)
    W2 = W2.at[:P, :D].set(W).at[P:, D:].set(W)
    outT = _tc_project_t(e128, W2)
    return outT.transpose(2, 0, 1)


# speedup vs baseline: 4.2229x; 1.0901x over previous
import functools

import jax
import jax.numpy as jnp
from jax import lax
from jax.experimental import pallas as pl
from jax.experimental.pallas import tpu as pltpu
from jax.experimental.pallas import tpu_sc as plsc

B = 16384
L = 50
FLAT = B * L
D = 64
P = 64

NC = 2
NS = 16
NW = NC * NS
G = 128
GROUPS = FLAT // G
GPW = GROUPS // NW
INNER = 8
OUTER = GPW // INNER


ROWS_PER_IT = INNER * G
HALF_IT = ROWS_PER_IT // 2


def _sc_gather(idxt, table):
    mesh = plsc.VectorSubcoreMesh(core_axis_name="c", subcore_axis_name="s")

    @functools.partial(
        pl.kernel,
        out_type=jax.ShapeDtypeStruct((FLAT, D), jnp.float32),
        mesh=mesh,
        compiler_params=pltpu.CompilerParams(use_tc_tiling_on_sc=False),
        scratch_types=[
            pltpu.VMEM((ROWS_PER_IT,), jnp.int32),
            pltpu.VMEM((ROWS_PER_IT,), jnp.int32),
            pltpu.VMEM((ROWS_PER_IT, D), jnp.float32),
            pltpu.SemaphoreType.DMA,
        ],
    )
    def k(idx_hbm, table_hbm, out_hbm, idx_ab, idx_v, rows_v, sem):
        wid = lax.axis_index("s") * NC + lax.axis_index("c")
        r_base = wid * (FLAT // NW)

        def body(c, _):
            lane = lax.iota(jnp.int32, 16)
            q0 = lax.shift_right_logical(lane, 1)
            evenm = (lane & 1) == 0
            r0 = r_base + c * ROWS_PER_IT
            l = lax.shift_right_logical(r0, 14)
            k0 = pl.multiple_of(
                lax.shift_right_logical(r0 - l * B, 1), HALF_IT)
            pltpu.sync_copy(idx_hbm.at[l, pl.ds(k0, HALF_IT)],
                            idx_ab.at[pl.ds(0, HALF_IT)])
            pltpu.sync_copy(idx_hbm.at[l, pl.ds(B // 2 + k0, HALF_IT)],
                            idx_ab.at[pl.ds(HALF_IT, HALF_IT)])
            for m in range(HALF_IT // 16):
                a = idx_ab[pl.ds(m * 16, 16)]
                b = idx_ab[pl.ds(HALF_IT + m * 16, 16)]
                lo = jnp.where(evenm, a.at[q0].get(mode='promise_in_bounds'),
                               b.at[q0].get(mode='promise_in_bounds'))
                hi = jnp.where(evenm,
                               a.at[q0 + 8].get(mode='promise_in_bounds'),
                               b.at[q0 + 8].get(mode='promise_in_bounds'))
                idx_v[pl.ds(m * 32, 16)] = lo
                idx_v[pl.ds(m * 32 + 16, 16)] = hi
            copies = [
                pltpu.async_copy(
                    table_hbm.at[idx_v.at[pl.ds(j * G, G)]],
                    rows_v.at[pl.ds(j * G, G)],
                    sem,
                )
                for j in range(INNER)
            ]
            for cp in copies:
                cp.wait()
            pltpu.sync_copy(rows_v, out_hbm.at[pl.ds(r0, ROWS_PER_IT)])
            return _

        lax.fori_loop(0, OUTER, body, None)

    return k(idxt, table)


SPLIT = 524288
NUMR = 1000000
PACK_BLK = 16384
PACK_NB = SPLIT // PACK_BLK
MAX_LANE_BLK = (NUMR + PACK_BLK - 1) // PACK_BLK - 1


def _tc_pack(tableT, ident):
    def body(lo_ref, hi_ref, i_ref, o_ref):
        tlo = lax.dot_general(
            lo_ref[...], i_ref[...],
            dimension_numbers=(((0,), (0,)), ((), ())),
            preferred_element_type=jnp.float32,
        )
        thi = lax.dot_general(
            hi_ref[...], i_ref[...],
            dimension_numbers=(((0,), (0,)), ((), ())),
            preferred_element_type=jnp.float32,
        )
        o_ref[:, :D] = tlo
        o_ref[:, D:] = thi

    return pl.pallas_call(
        body,
        grid=(PACK_NB,),
        in_specs=[
            pl.BlockSpec((D, PACK_BLK), lambda i: (0, i)),
            pl.BlockSpec((D, PACK_BLK),
                         lambda i: (0, jnp.minimum(PACK_NB + i, MAX_LANE_BLK))),
            pl.BlockSpec((D, D), lambda i: (0, 0)),
        ],
        out_specs=pl.BlockSpec((PACK_BLK, 2 * D), lambda i: (i, 0)),
        out_shape=jax.ShapeDtypeStruct((SPLIT, 2 * D), jnp.float32),
    )(tableT, tableT, ident)


def _tc_project_t(e128, W2):
    H = B // 2

    def body(e_ref, w_ref, o_ref):
        r2 = lax.dot_general(
            w_ref[...], e_ref[...],
            dimension_numbers=(((1,), (1,)), ((), ())),
            preferred_element_type=jnp.float32,
        )
        o_ref[0, :, :H] = r2[:P, :]
        o_ref[0, :, H:] = r2[P:, :]

    return pl.pallas_call(
        body,
        grid=(L,),
        in_specs=[
            pl.BlockSpec((H, 2 * D), lambda i: (i, 0)),
            pl.BlockSpec((2 * P, 2 * D), lambda i: (0, 0)),
        ],
        out_specs=pl.BlockSpec((1, P, B), lambda i: (i, 0, 0)),
        out_shape=jax.ShapeDtypeStruct((L, P, B), jnp.float32),
    )(e128, W2)


def kernel(indexes, table, W):
    idxt = indexes.astype(jnp.int32).T
    idxt = jnp.where(idxt < SPLIT, idxt * 2, (idxt - SPLIT) * 2 + 1)
    packed = _tc_pack(table.T, jnp.eye(D, dtype=jnp.float32))
    pview = packed.reshape(2 * SPLIT, D)
    emb = _sc_gather(idxt, pview)
    e128 = emb.reshape(FLAT // 2, 2 * D)
    W2 = jnp.zeros((2 * P, 2 * D), jnp.float32)
    W2 = W2.at[:P, :D].set(W).at[P:, D:].set(W)
    outT = _tc_project_t(e128, W2)
    return outT.transpose(2, 0, 1)


# speedup vs baseline: 4.3479x; 1.0296x over previous
import functools

import jax
import jax.numpy as jnp
from jax import lax
from jax.experimental import pallas as pl
from jax.experimental.pallas import tpu as pltpu
from jax.experimental.pallas import tpu_sc as plsc

B = 16384
L = 50
FLAT = B * L
D = 64
P = 64

NC = 2
NS = 16
NW = NC * NS
G = 128
GROUPS = FLAT // G
GPW = GROUPS // NW
INNER = 8
OUTER = GPW // INNER


RI = 512
HALF_IT = RI // 2
NG = RI // G
NBLK = FLAT // NW // RI


def _sc_gather(idxt, table):
    mesh = plsc.VectorSubcoreMesh(core_axis_name="c", subcore_axis_name="s")

    @functools.partial(
        pl.kernel,
        out_type=jax.ShapeDtypeStruct((FLAT, D), jnp.float32),
        mesh=mesh,
        compiler_params=pltpu.CompilerParams(use_tc_tiling_on_sc=False),
        scratch_types=[
            pltpu.VMEM((RI,), jnp.int32),
            pltpu.VMEM((RI,), jnp.int32),
            pltpu.VMEM((RI,), jnp.int32),
            pltpu.VMEM((RI, D), jnp.float32),
            pltpu.VMEM((RI, D), jnp.float32),
            pltpu.SemaphoreType.DMA,
            pltpu.SemaphoreType.DMA,
        ],
    )
    def k(idx_hbm, table_hbm, out_hbm, idx_ab, idx_v0, idx_v1,
          rows0, rows1, sem_g, sem_s):
        wid = lax.axis_index("s") * NC + lax.axis_index("c")
        r_base = wid * (FLAT // NW)

        def prepare(c, idx_v):
            lane = lax.iota(jnp.int32, 16)
            q0 = lax.shift_right_logical(lane, 1)
            evenm = (lane & 1) == 0
            r0 = r_base + c * RI
            l = lax.shift_right_logical(r0, 14)
            k0 = pl.multiple_of(
                lax.shift_right_logical(r0 - l * B, 1), HALF_IT)
            pltpu.sync_copy(idx_hbm.at[l, pl.ds(k0, HALF_IT)],
                            idx_ab.at[pl.ds(0, HALF_IT)])
            pltpu.sync_copy(idx_hbm.at[l, pl.ds(B // 2 + k0, HALF_IT)],
                            idx_ab.at[pl.ds(HALF_IT, HALF_IT)])
            for m in range(HALF_IT // 16):
                a = idx_ab[pl.ds(m * 16, 16)]
                b = idx_ab[pl.ds(HALF_IT + m * 16, 16)]
                lo = jnp.where(evenm, a.at[q0].get(mode='promise_in_bounds'),
                               b.at[q0].get(mode='promise_in_bounds'))
                hi = jnp.where(evenm,
                               a.at[q0 + 8].get(mode='promise_in_bounds'),
                               b.at[q0 + 8].get(mode='promise_in_bounds'))
                idx_v[pl.ds(m * 32, 16)] = lo
                idx_v[pl.ds(m * 32 + 16, 16)] = hi

        def fire(idx_v, rows):
            return [
                pltpu.async_copy(
                    table_hbm.at[idx_v.at[pl.ds(j * G, G)]],
                    rows.at[pl.ds(j * G, G)],
                    sem_g,
                )
                for j in range(NG)
            ]

        def store(c, rows):
            r0 = r_base + c * RI
            pltpu.async_copy(rows, out_hbm.at[pl.ds(r0, RI)], sem_s)

        def drain_store(rows):
            pltpu.make_async_copy(
                out_hbm.at[pl.ds(r_base, RI)], rows, sem_s).wait()

        def body(t, _):
            c0 = t * 2
            c1 = c0 + 1

            @pl.when(t > 0)
            def _d0():
                drain_store(rows0)

            prepare(c0, idx_v0)
            g0 = fire(idx_v0, rows0)

            @pl.when(t > 0)
            def _d1():
                drain_store(rows1)

            prepare(c1, idx_v1)
            for cp in g0:
                cp.wait()
            store(c0, rows0)
            g1 = fire(idx_v1, rows1)
            for cp in g1:
                cp.wait()
            store(c1, rows1)
            return _

        lax.fori_loop(0, NBLK // 2, body, None)
        drain_store(rows0)
        drain_store(rows1)

    return k(idxt, table)


SPLIT = 524288
NUMR = 1000000
PACK_BLK = 16384
PACK_NB = SPLIT // PACK_BLK
MAX_LANE_BLK = (NUMR + PACK_BLK - 1) // PACK_BLK - 1


def _tc_pack(tableT, ident):
    def body(lo_ref, hi_ref, i_ref, o_ref):
        tlo = lax.dot_general(
            lo_ref[...], i_ref[...],
            dimension_numbers=(((0,), (0,)), ((), ())),
            preferred_element_type=jnp.float32,
        )
        thi = lax.dot_general(
            hi_ref[...], i_ref[...],
            dimension_numbers=(((0,), (0,)), ((), ())),
            preferred_element_type=jnp.float32,
        )
        o_ref[:, :D] = tlo
        o_ref[:, D:] = thi

    return pl.pallas_call(
        body,
        grid=(PACK_NB,),
        in_specs=[
            pl.BlockSpec((D, PACK_BLK), lambda i: (0, i)),
            pl.BlockSpec((D, PACK_BLK),
                         lambda i: (0, jnp.minimum(PACK_NB + i, MAX_LANE_BLK))),
            pl.BlockSpec((D, D), lambda i: (0, 0)),
        ],
        out_specs=pl.BlockSpec((PACK_BLK, 2 * D), lambda i: (i, 0)),
        out_shape=jax.ShapeDtypeStruct((SPLIT, 2 * D), jnp.float32),
    )(tableT, tableT, ident)


def _tc_project_t(e128, W2):
    H = B // 2

    def body(e_ref, w_ref, o_ref):
        r2 = lax.dot_general(
            w_ref[...], e_ref[...],
            dimension_numbers=(((1,), (1,)), ((), ())),
            preferred_element_type=jnp.float32,
        )
        o_ref[0, :, :H] = r2[:P, :]
        o_ref[0, :, H:] = r2[P:, :]

    return pl.pallas_call(
        body,
        grid=(L,),
        in_specs=[
            pl.BlockSpec((H, 2 * D), lambda i: (i, 0)),
            pl.BlockSpec((2 * P, 2 * D), lambda i: (0, 0)),
        ],
        out_specs=pl.BlockSpec((1, P, B), lambda i: (i, 0, 0)),
        out_shape=jax.ShapeDtypeStruct((L, P, B), jnp.float32),
    )(e128, W2)


def kernel(indexes, table, W):
    idxt = indexes.astype(jnp.int32).T
    idxt = jnp.where(idxt < SPLIT, idxt * 2, (idxt - SPLIT) * 2 + 1)
    packed = _tc_pack(table.T, jnp.eye(D, dtype=jnp.float32))
    pview = packed.reshape(2 * SPLIT, D)
    emb = _sc_gather(idxt, pview)
    e128 = emb.reshape(FLAT // 2, 2 * D)
    W2 = jnp.zeros((2 * P, 2 * D), jnp.float32)
    W2 = W2.at[:P, :D].set(W).at[P:, D:].set(W)
    outT = _tc_project_t(e128, W2)
    return outT.transpose(2, 0, 1)
